# baseline (speedup 1.0000x reference)
import functools

import jax
import jax.numpy as jnp
from jax import lax
from jax.experimental import pallas as pl
from jax.experimental.pallas import tpu as pltpu
from jax.experimental.pallas import tpu_sc as plsc

VOCAB = 100000
NDIM = 64
NNEG = 5

NC = 2
NS = 16
NW = NC * NS

CHUNK = 128
NB = 2


def _emit_phase(emb, idx_hbm, out_hbm, base_w, n_chunks,
                idx_v, rows_v, isem, gsem, wsem):
    def start_idx(b, c):
        pltpu.async_copy(idx_hbm.at[pl.ds(base_w + c * CHUNK, CHUNK)],
                         idx_v[b], isem[b])

    def wait_idx(b):
        pltpu.make_async_copy(idx_hbm.at[pl.ds(0, CHUNK)], idx_v[b],
                              isem[b]).wait()

    def start_gather(b):
        pltpu.async_copy(emb.at[idx_v[b]], rows_v[b], gsem[b])

    def wait_gather(b):
        pltpu.make_async_copy(emb.at[idx_v[b]], rows_v[b], gsem[b]).wait()

    def start_write(b, c):
        pltpu.async_copy(rows_v[b],
                         out_hbm.at[pl.ds(base_w + c * CHUNK, CHUNK)], wsem[b])

    def wait_write(b):
        pltpu.make_async_copy(rows_v[b], out_hbm.at[pl.ds(0, CHUNK)],
                              wsem[b]).wait()

    if n_chunks < 2 * NB:
        for c in range(n_chunks):
            b = c % NB
            start_idx(b, c)
            wait_idx(b)
            start_gather(b)
            wait_gather(b)
            start_write(b, c)
        for c in range(max(0, n_chunks - NB), n_chunks):
            wait_write(c % NB)
        return

    assert n_chunks % NB == 0
    n_groups = n_chunks // NB

    def group_body(g, *, first, last):
        for b in range(NB):
            c = g * NB + b
            wait_gather(b)
            if not last:
                start_idx(b, c + NB)
            start_write(b, c)
            if last and b == NB - 1:
                continue
            b1 = (b + 1) % NB
            c1_ge_nb = (not first) or (b + 1 >= NB)
            if c1_ge_nb:
                wait_write(b1)
            wait_idx(b1)
            start_gather(b1)

    for b in range(NB):
        start_idx(b, b)
    wait_idx(0)
    start_gather(0)

    group_body(0, first=True, last=(n_groups == 1))
    if n_groups > 2:
        def loop_body(g, carry):
            group_body(g, first=False, last=False)
            return carry
        lax.fori_loop(1, n_groups - 1, loop_body, 0)
    if n_groups > 1:
        group_body(n_groups - 1, first=False, last=True)

    for b in range(NB):
        wait_write(b)


def _make_gather(n_all):
    mesh = plsc.VectorSubcoreMesh(core_axis_name="c", subcore_axis_name="s")

    @functools.partial(
        pl.kernel,
        mesh=mesh,
        out_type=jax.ShapeDtypeStruct((n_all, NDIM), jnp.float32),
        scratch_types=[
            pltpu.VMEM((CHUNK,), jnp.int32),
            pltpu.VMEM((CHUNK,), jnp.int32),
            pltpu.VMEM((CHUNK, NDIM), jnp.float32),
            pltpu.VMEM((CHUNK, NDIM), jnp.float32),
            pltpu.SemaphoreType.DMA,
            pltpu.SemaphoreType.DMA,
            pltpu.SemaphoreType.DMA,
            pltpu.SemaphoreType.DMA,
            pltpu.SemaphoreType.DMA,
            pltpu.SemaphoreType.DMA,
        ],
        compiler_params=pltpu.CompilerParams(use_tc_tiling_on_sc=False),
    )
    def gather_kernel(all_idx, emb, out,
                      idx0, idx1, rows0, rows1,
                      isem0, isem1, gsem0, gsem1, wsem0, wsem1):
        wid = lax.axis_index("s") * NC + lax.axis_index("c")
        per_w = n_all // NW
        _emit_phase(emb, all_idx, out, wid * per_w, per_w // CHUNK,
                    [idx0, idx1], [rows0, rows1],
                    [isem0, isem1], [gsem0, gsem1], [wsem0, wsem1])

    return gather_kernel


def kernel(contexts, centers, embedding):
    n_center = centers.shape[0]
    win = contexts.shape[1]
    n_ctx = n_center * win
    n_noi = n_center * NNEG * win

    noise_key = jax.random.fold_in(jax.random.key(0), 12345)
    noise_words = jax.random.randint(noise_key, (n_noi,), 0, VOCAB)

    n_all = n_ctx + n_center + n_noi
    pad = (-n_all) % (NW * CHUNK * NB)
    all_idx = jnp.concatenate([
        contexts.reshape(-1).astype(jnp.int32),
        centers.astype(jnp.int32),
        noise_words.astype(jnp.int32),
        jnp.zeros((pad,), jnp.int32),
    ])

    out = _make_gather(all_idx.shape[0])(all_idx, embedding)
    return (
        out[:n_ctx].reshape(n_center, win, NDIM),
        out[n_ctx:n_ctx + n_center],
        out[n_ctx + n_center:n_all].reshape(n_center, win * NNEG, NDIM),
    )


# speedup vs baseline: 2.1639x; 2.1639x over previous
import functools

import jax
import jax.numpy as jnp
from jax import lax
from jax.experimental import pallas as pl
from jax.experimental.pallas import tpu as pltpu
from jax.experimental.pallas import tpu_sc as plsc

VOCAB = 100000
NDIM = 64
NNEG = 5

NC = 2
NS = 16
NW = NC * NS

CHUNK_BIG = 320
CHUNK_CEN = 128
NB = 2


def _emit_phase(emb, idx_hbm, out_hbm, base_w, n_chunks, CHUNK,
                idx_v, rows_v, isem, gsem, wsem):
    def start_idx(b, c):
        pltpu.async_copy(idx_hbm.at[pl.ds(base_w + c * CHUNK, CHUNK)],
                         idx_v[b], isem[b])

    def wait_idx(b):
        pltpu.make_async_copy(idx_hbm.at[pl.ds(0, CHUNK)], idx_v[b],
                              isem[b]).wait()

    def start_gather(b):
        pltpu.async_copy(emb.at[idx_v[b]], rows_v[b], gsem[b])

    def wait_gather(b):
        pltpu.make_async_copy(emb.at[idx_v[b]], rows_v[b], gsem[b]).wait()

    def start_write(b, c):
        pltpu.async_copy(rows_v[b],
                         out_hbm.at[pl.ds(base_w + c * CHUNK, CHUNK)], wsem[b])

    def wait_write(b):
        pltpu.make_async_copy(rows_v[b], out_hbm.at[pl.ds(0, CHUNK)],
                              wsem[b]).wait()

    if n_chunks < 2 * NB:
        for c in range(n_chunks):
            b = c % NB
            start_idx(b, c)
            wait_idx(b)
            start_gather(b)
            wait_gather(b)
            start_write(b, c)
        for c in range(max(0, n_chunks - NB), n_chunks):
            wait_write(c % NB)
        return

    assert n_chunks % NB == 0
    n_groups = n_chunks // NB

    def group_body(g, *, first, last):
        for b in range(NB):
            c = g * NB + b
            wait_gather(b)
            if not last:
                start_idx(b, c + NB)
            start_write(b, c)
            if last and b == NB - 1:
                continue
            b1 = (b + 1) % NB
            c1_ge_nb = (not first) or (b + 1 >= NB)
            if c1_ge_nb:
                wait_write(b1)
            wait_idx(b1)
            start_gather(b1)

    for b in range(NB):
        start_idx(b, b)
    wait_idx(0)
    start_gather(0)

    group_body(0, first=True, last=(n_groups == 1))
    if n_groups > 2:
        def loop_body(g, carry):
            group_body(g, first=False, last=False)
            return carry
        lax.fori_loop(1, n_groups - 1, loop_body, 0)
    if n_groups > 1:
        group_body(n_groups - 1, first=False, last=True)

    for b in range(NB):
        wait_write(b)


def _make_gather(n_ctx, n_cen, n_noi):
    mesh = plsc.VectorSubcoreMesh(core_axis_name="c", subcore_axis_name="s")

    @functools.partial(
        pl.kernel,
        mesh=mesh,
        out_type=(
            jax.ShapeDtypeStruct((n_ctx, NDIM), jnp.float32),
            jax.ShapeDtypeStruct((n_cen, NDIM), jnp.float32),
            jax.ShapeDtypeStruct((n_noi, NDIM), jnp.float32),
        ),
        scratch_types=[
            pltpu.VMEM((CHUNK_BIG,), jnp.int32),
            pltpu.VMEM((CHUNK_BIG,), jnp.int32),
            pltpu.VMEM((CHUNK_BIG, NDIM), jnp.float32),
            pltpu.VMEM((CHUNK_BIG, NDIM), jnp.float32),
            pltpu.SemaphoreType.DMA,
            pltpu.SemaphoreType.DMA,
            pltpu.SemaphoreType.DMA,
            pltpu.SemaphoreType.DMA,
            pltpu.SemaphoreType.DMA,
            pltpu.SemaphoreType.DMA,
        ],
        compiler_params=pltpu.CompilerParams(use_tc_tiling_on_sc=False),
    )
    def gather_kernel(ctx_idx, cen_idx, noi_idx, emb,
                      out_ctx, out_cen, out_noi,
                      idx0, idx1, rows0, rows1,
                      isem0, isem1, gsem0, gsem1, wsem0, wsem1):
        wid = lax.axis_index("s") * NC + lax.axis_index("c")
        isem = [isem0, isem1]
        gsem = [gsem0, gsem1]
        wsem = [wsem0, wsem1]
        for idx_hbm, out_hbm, n_total, chunk in (
            (ctx_idx, out_ctx, n_ctx, CHUNK_BIG),
            (cen_idx, out_cen, n_cen, CHUNK_CEN),
            (noi_idx, out_noi, n_noi, CHUNK_BIG),
        ):
            idx_v = [idx0.at[pl.ds(0, chunk)], idx1.at[pl.ds(0, chunk)]]
            rows_v = [rows0.at[pl.ds(0, chunk), :], rows1.at[pl.ds(0, chunk), :]]
            per_w = n_total // NW
            _emit_phase(emb, idx_hbm, out_hbm, wid * per_w, per_w // chunk,
                        chunk, idx_v, rows_v, isem, gsem, wsem)

    return gather_kernel


def kernel(contexts, centers, embedding):
    n_center = centers.shape[0]
    win = contexts.shape[1]
    n_ctx = n_center * win
    n_noi = n_center * NNEG * win

    noise_key = jax.random.fold_in(jax.random.key(0), 12345)
    noise_words = jax.random.randint(noise_key, (n_noi,), 0, VOCAB)

    ctx_idx = contexts.reshape(-1).astype(jnp.int32)
    cen_idx = centers.astype(jnp.int32)
    noi_idx = noise_words.astype(jnp.int32)

    out_ctx, out_cen, out_noi = _make_gather(n_ctx, n_center, n_noi)(
        ctx_idx, cen_idx, noi_idx, embedding
    )
    return (
        out_ctx.reshape(n_center, win, NDIM),
        out_cen,
        out_noi.reshape(n_center, win * NNEG, NDIM),
    )


# speedup vs baseline: 2.1935x; 1.0137x over previous
import functools

import jax
import jax.numpy as jnp
from jax import lax
from jax.experimental import pallas as pl
from jax.experimental.pallas import tpu as pltpu
from jax.experimental.pallas import tpu_sc as plsc

VOCAB = 100000
NDIM = 64
NNEG = 5

NC = 2
NS = 16
NW = NC * NS

CHUNK_NOI = 640
CHUNK_BIG = 320
CHUNK_CEN = 128
NB = 2


def _emit_phase(emb, idx_hbm, out_hbm, base_w, n_chunks, CHUNK,
                idx_v, rows_v, isem, gsem, wsem):
    def start_idx(b, c):
        pltpu.async_copy(idx_hbm.at[pl.ds(base_w + c * CHUNK, CHUNK)],
                         idx_v[b], isem[b])

    def wait_idx(b):
        pltpu.make_async_copy(idx_hbm.at[pl.ds(0, CHUNK)], idx_v[b],
                              isem[b]).wait()

    def start_gather(b):
        pltpu.async_copy(emb.at[idx_v[b]], rows_v[b], gsem[b])

    def wait_gather(b):
        pltpu.make_async_copy(emb.at[idx_v[b]], rows_v[b], gsem[b]).wait()

    def start_write(b, c):
        pltpu.async_copy(rows_v[b],
                         out_hbm.at[pl.ds(base_w + c * CHUNK, CHUNK)], wsem[b])

    def wait_write(b):
        pltpu.make_async_copy(rows_v[b], out_hbm.at[pl.ds(0, CHUNK)],
                              wsem[b]).wait()

    if n_chunks < 2 * NB:
        for c in range(n_chunks):
            b = c % NB
            start_idx(b, c)
            wait_idx(b)
            start_gather(b)
            wait_gather(b)
            start_write(b, c)
        for c in range(max(0, n_chunks - NB), n_chunks):
            wait_write(c % NB)
        return

    assert n_chunks % NB == 0
    n_groups = n_chunks // NB

    def group_body(g, *, first, last):
        for b in range(NB):
            c = g * NB + b
            wait_gather(b)
            if not last:
                start_idx(b, c + NB)
            start_write(b, c)
            if last and b == NB - 1:
                continue
            b1 = (b + 1) % NB
            c1_ge_nb = (not first) or (b + 1 >= NB)
            if c1_ge_nb:
                wait_write(b1)
            wait_idx(b1)
            start_gather(b1)

    for b in range(NB):
        start_idx(b, b)
    wait_idx(0)
    start_gather(0)

    group_body(0, first=True, last=(n_groups == 1))
    if n_groups > 2:
        def loop_body(g, carry):
            group_body(g, first=False, last=False)
            return carry
        lax.fori_loop(1, n_groups - 1, loop_body, 0)
    if n_groups > 1:
        group_body(n_groups - 1, first=False, last=True)

    for b in range(NB):
        wait_write(b)


def _make_gather(n_ctx, n_cen, n_noi):
    mesh = plsc.VectorSubcoreMesh(core_axis_name="c", subcore_axis_name="s")

    @functools.partial(
        pl.kernel,
        mesh=mesh,
        out_type=(
            jax.ShapeDtypeStruct((n_ctx, NDIM), jnp.float32),
            jax.ShapeDtypeStruct((n_cen, NDIM), jnp.float32),
            jax.ShapeDtypeStruct((n_noi, NDIM), jnp.float32),
        ),
        scratch_types=[
            pltpu.VMEM((CHUNK_NOI,), jnp.int32),
            pltpu.VMEM((CHUNK_NOI,), jnp.int32),
            pltpu.VMEM((CHUNK_NOI, NDIM), jnp.float32),
            pltpu.VMEM((CHUNK_NOI, NDIM), jnp.float32),
            pltpu.SemaphoreType.DMA,
            pltpu.SemaphoreType.DMA,
            pltpu.SemaphoreType.DMA,
            pltpu.SemaphoreType.DMA,
            pltpu.SemaphoreType.DMA,
            pltpu.SemaphoreType.DMA,
        ],
        compiler_params=pltpu.CompilerParams(use_tc_tiling_on_sc=False),
    )
    def gather_kernel(ctx_idx, cen_idx, noi_idx, emb,
                      out_ctx, out_cen, out_noi,
                      idx0, idx1, rows0, rows1,
                      isem0, isem1, gsem0, gsem1, wsem0, wsem1):
        wid = lax.axis_index("s") * NC + lax.axis_index("c")
        isem = [isem0, isem1]
        gsem = [gsem0, gsem1]
        wsem = [wsem0, wsem1]
        for idx_hbm, out_hbm, n_total, chunk in (
            (ctx_idx, out_ctx, n_ctx, CHUNK_BIG),
            (cen_idx, out_cen, n_cen, CHUNK_CEN),
            (noi_idx, out_noi, n_noi, CHUNK_NOI),
        ):
            idx_v = [idx0.at[pl.ds(0, chunk)], idx1.at[pl.ds(0, chunk)]]
            rows_v = [rows0.at[pl.ds(0, chunk), :], rows1.at[pl.ds(0, chunk), :]]
            per_w = n_total // NW
            _emit_phase(emb, idx_hbm, out_hbm, wid * per_w, per_w // chunk,
                        chunk, idx_v, rows_v, isem, gsem, wsem)

    return gather_kernel


def kernel(contexts, centers, embedding):
    n_center = centers.shape[0]
    win = contexts.shape[1]
    n_ctx = n_center * win
    n_noi = n_center * NNEG * win

    noise_key = jax.random.fold_in(jax.random.key(0), 12345)
    noise_words = jax.random.randint(noise_key, (n_noi,), 0, VOCAB)

    ctx_idx = contexts.reshape(-1).astype(jnp.int32)
    cen_idx = centers.astype(jnp.int32)
    noi_idx = noise_words.astype(jnp.int32)

    out_ctx, out_cen, out_noi = _make_gather(n_ctx, n_center, n_noi)(
        ctx_idx, cen_idx, noi_idx, embedding
    )
    return (
        out_ctx.reshape(n_center, win, NDIM),
        out_cen,
        out_noi.reshape(n_center, win * NNEG, NDIM),
    )


# speedup vs baseline: 2.1976x; 1.0019x over previous
import functools

import jax
import jax.numpy as jnp
from jax import lax
from jax.experimental import pallas as pl
from jax.experimental.pallas import tpu as pltpu
from jax.experimental.pallas import tpu_sc as plsc

VOCAB = 100000
NDIM = 64
NNEG = 5

NC = 2
NS = 16
NW = NC * NS

CHUNK_NOI = 800
CHUNK_BIG = 320
CHUNK_CEN = 128
NB = 2


def _emit_phase(emb, idx_hbm, out_hbm, base_w, n_chunks, CHUNK,
                idx_v, rows_v, isem, gsem, wsem):
    def start_idx(b, c):
        pltpu.async_copy(idx_hbm.at[pl.ds(base_w + c * CHUNK, CHUNK)],
                         idx_v[b], isem[b])

    def wait_idx(b):
        pltpu.make_async_copy(idx_hbm.at[pl.ds(0, CHUNK)], idx_v[b],
                              isem[b]).wait()

    def start_gather(b):
        pltpu.async_copy(emb.at[idx_v[b]], rows_v[b], gsem[b])

    def wait_gather(b):
        pltpu.make_async_copy(emb.at[idx_v[b]], rows_v[b], gsem[b]).wait()

    def start_write(b, c):
        pltpu.async_copy(rows_v[b],
                         out_hbm.at[pl.ds(base_w + c * CHUNK, CHUNK)], wsem[b])

    def wait_write(b):
        pltpu.make_async_copy(rows_v[b], out_hbm.at[pl.ds(0, CHUNK)],
                              wsem[b]).wait()

    if n_chunks < 2 * NB:
        for c in range(n_chunks):
            b = c % NB
            start_idx(b, c)
            wait_idx(b)
            start_gather(b)
            wait_gather(b)
            start_write(b, c)
        for c in range(max(0, n_chunks - NB), n_chunks):
            wait_write(c % NB)
        return

    assert n_chunks % NB == 0
    n_groups = n_chunks // NB

    def group_body(g, *, first, last):
        for b in range(NB):
            c = g * NB + b
            wait_gather(b)
            if not last:
                start_idx(b, c + NB)
            start_write(b, c)
            if last and b == NB - 1:
                continue
            b1 = (b + 1) % NB
            c1_ge_nb = (not first) or (b + 1 >= NB)
            if c1_ge_nb:
                wait_write(b1)
            wait_idx(b1)
            start_gather(b1)

    for b in range(NB):
        start_idx(b, b)
    wait_idx(0)
    start_gather(0)

    group_body(0, first=True, last=(n_groups == 1))
    if n_groups > 2:
        def loop_body(g, carry):
            group_body(g, first=False, last=False)
            return carry
        lax.fori_loop(1, n_groups - 1, loop_body, 0)
    if n_groups > 1:
        group_body(n_groups - 1, first=False, last=True)

    for b in range(NB):
        wait_write(b)


def _make_gather(n_ctx, n_cen, n_noi):
    mesh = plsc.VectorSubcoreMesh(core_axis_name="c", subcore_axis_name="s")

    @functools.partial(
        pl.kernel,
        mesh=mesh,
        out_type=(
            jax.ShapeDtypeStruct((n_ctx, NDIM), jnp.float32),
            jax.ShapeDtypeStruct((n_cen, NDIM), jnp.float32),
            jax.ShapeDtypeStruct((n_noi, NDIM), jnp.float32),
        ),
        scratch_types=[
            pltpu.VMEM((CHUNK_NOI,), jnp.int32),
            pltpu.VMEM((CHUNK_NOI,), jnp.int32),
            pltpu.VMEM((CHUNK_NOI, NDIM), jnp.float32),
            pltpu.VMEM((CHUNK_NOI, NDIM), jnp.float32),
            pltpu.SemaphoreType.DMA,
            pltpu.SemaphoreType.DMA,
            pltpu.SemaphoreType.DMA,
            pltpu.SemaphoreType.DMA,
            pltpu.SemaphoreType.DMA,
            pltpu.SemaphoreType.DMA,
        ],
        compiler_params=pltpu.CompilerParams(use_tc_tiling_on_sc=False),
    )
    def gather_kernel(ctx_idx, cen_idx, noi_idx, emb,
                      out_ctx, out_cen, out_noi,
                      idx0, idx1, rows0, rows1,
                      isem0, isem1, gsem0, gsem1, wsem0, wsem1):
        wid = lax.axis_index("s") * NC + lax.axis_index("c")
        isem = [isem0, isem1]
        gsem = [gsem0, gsem1]
        wsem = [wsem0, wsem1]
        for idx_hbm, out_hbm, n_total, chunk in (
            (ctx_idx, out_ctx, n_ctx, CHUNK_BIG),
            (cen_idx, out_cen, n_cen, CHUNK_CEN),
            (noi_idx, out_noi, n_noi, CHUNK_NOI),
        ):
            idx_v = [idx0.at[pl.ds(0, chunk)], idx1.at[pl.ds(0, chunk)]]
            rows_v = [rows0.at[pl.ds(0, chunk), :], rows1.at[pl.ds(0, chunk), :]]
            per_w = n_total // NW
            _emit_phase(emb, idx_hbm, out_hbm, wid * per_w, per_w // chunk,
                        chunk, idx_v, rows_v, isem, gsem, wsem)

    return gather_kernel


def kernel(contexts, centers, embedding):
    n_center = centers.shape[0]
    win = contexts.shape[1]
    n_ctx = n_center * win
    n_noi = n_center * NNEG * win

    noise_key = jax.random.fold_in(jax.random.key(0), 12345)
    noise_words = jax.random.randint(noise_key, (n_noi,), 0, VOCAB)

    ctx_idx = contexts.reshape(-1).astype(jnp.int32)
    cen_idx = centers.astype(jnp.int32)
    noi_idx = noise_words.astype(jnp.int32)

    out_ctx, out_cen, out_noi = _make_gather(n_ctx, n_center, n_noi)(
        ctx_idx, cen_idx, noi_idx, embedding
    )
    return (
        out_ctx.reshape(n_center, win, NDIM),
        out_cen,
        out_noi.reshape(n_center, win * NNEG, NDIM),
    )
